# baseline (speedup 1.0000x reference)
import functools

import jax
import jax.numpy as jnp
from jax import lax
from jax.experimental import pallas as pl
from jax.experimental.pallas import tpu as pltpu
from jax.experimental.pallas import tpu_sc as plsc

_BATCH = 4096
_HIST = 50
_EMBED = 64
_B = _BATCH * _HIST

_NC = 2
_NS = 16
_NW = _NC * _NS
_B_PER_W = _B // _NW
_CHUNK = 800
_NCHUNK = _B_PER_W // _CHUNK
_NBUF = 2

_mesh = plsc.VectorSubcoreMesh(core_axis_name="c", subcore_axis_name="s")


@functools.partial(
    pl.kernel,
    mesh=_mesh,
    out_type=jax.ShapeDtypeStruct((_B, _EMBED), jnp.float32),
    compiler_params=pltpu.CompilerParams(use_tc_tiling_on_sc=False),
    scratch_types=[
        pltpu.VMEM((_B_PER_W,), jnp.int32),
        pltpu.VMEM((_NBUF, _CHUNK, _EMBED), jnp.float32),
        pltpu.SemaphoreType.DMA,
        pltpu.SemaphoreType.DMA,
        pltpu.SemaphoreType.DMA,
        pltpu.SemaphoreType.DMA,
    ],
)
def _gather(idx_hbm, table_hbm, out_hbm, idx_v, rows_v, g0, g1, w0, w1):
    wid = lax.axis_index("s") * _NC + lax.axis_index("c")
    base = wid * _B_PER_W
    pltpu.sync_copy(idx_hbm.at[pl.ds(base, _B_PER_W)], idx_v)

    gsem = (g0, g1)
    wsem = (w0, w1)
    gathers = [None] * _NBUF
    writes = [None] * _NBUF
    for c in range(_NCHUNK + 1):
        if c < _NCHUNK:
            buf = c % _NBUF
            if writes[buf] is not None:
                writes[buf].wait()
                writes[buf] = None
            gathers[buf] = pltpu.async_copy(
                table_hbm.at[idx_v.at[pl.ds(c * _CHUNK, _CHUNK)]],
                rows_v.at[buf],
                gsem[buf],
            )
        if c > 0:
            pbuf = (c - 1) % _NBUF
            gathers[pbuf].wait()
            writes[pbuf] = pltpu.async_copy(
                rows_v.at[pbuf],
                out_hbm.at[pl.ds(base + (c - 1) * _CHUNK, _CHUNK)],
                wsem[pbuf],
            )
    for buf in range(_NBUF):
        if writes[buf] is not None:
            writes[buf].wait()


def kernel(inputs, table):
    idx = inputs.reshape(_B).astype(jnp.int32)
    out = _gather(idx, table)
    return out.reshape(_BATCH, _HIST, _EMBED)
